# Initial kernel scaffold; baseline (speedup 1.0000x reference)
import jax
import jax.numpy as jnp
from jax.experimental import pallas as pl


def kernel(x, edge_index, edge_attr, graph_attr, batch, node_W, node_b, edge_W, edge_b, c1_W1, c1_b1, c1_W2, c1_b2, c2_W1, c2_b1, c2_W2, c2_b2, c3_W1, c3_b1, c3_W2, c3_b2, d1_W, d1_b, d2_W, d2_b, o_W, o_b):
    raise NotImplementedError("write your pallas kernel here")


# speedup vs baseline: 1.9352x; 1.9352x over previous
import jax
import jax.numpy as jnp
from jax.experimental import pallas as pl
from jax.experimental.pallas import tpu as pltpu

N = 10000
E = 320000
D = 128
G = 16
EPS = 1e-7


def _head_body(pooled_ref, ga_ref, d1W_ref, d1b_ref, d2W_ref, d2b_ref, oW_ref, ob_ref, out_ref):
    g = jnp.concatenate([pooled_ref[...], ga_ref[...]], axis=1)
    g = jax.nn.relu(g @ d1W_ref[...].T + d1b_ref[...])
    g = jax.nn.relu(g @ d2W_ref[...].T + d2b_ref[...])
    out_ref[...] = jax.nn.sigmoid(g @ oW_ref[...].T + ob_ref[...])


def _gen_conv(x, src, dst, edge_attr, W1, b1, W2, b2):
    msg = jax.nn.relu(x[src] + edge_attr) + EPS
    p = jnp.exp(msg)
    s = jax.ops.segment_sum(p, dst, num_segments=N)
    w = jax.ops.segment_sum(msg * p, dst, num_segments=N)
    aggr = w / jnp.maximum(s, 1e-30)
    out = x + aggr
    h = jax.nn.relu(out @ W1.T + b1)
    return h @ W2.T + b2


def kernel(x, edge_index, edge_attr, graph_attr, batch, node_W, node_b, edge_W, edge_b, c1_W1, c1_b1, c1_W2, c1_b2, c2_W1, c2_b1, c2_W2, c2_b2, c3_W1, c3_b1, c3_W2, c3_b2, d1_W, d1_b, d2_W, d2_b, o_W, o_b):
    src, dst = edge_index[0], edge_index[1]
    ea = edge_attr @ edge_W.T + edge_b
    h = x @ node_W.T + node_b
    h = jax.nn.relu(_gen_conv(h, src, dst, ea, c1_W1, c1_b1, c1_W2, c1_b2))
    h = jax.nn.relu(_gen_conv(h, src, dst, ea, c2_W1, c2_b1, c2_W2, c2_b2))
    h = jax.nn.relu(_gen_conv(h, src, dst, ea, c3_W1, c3_b1, c3_W2, c3_b2))
    ones = jnp.ones((N,), dtype=jnp.float32)
    counts = jax.ops.segment_sum(ones, batch, num_segments=G)
    pooled = jax.ops.segment_sum(h, batch, num_segments=G) / jnp.maximum(counts, 1.0)[:, None]
    out = pl.pallas_call(
        _head_body,
        out_shape=jax.ShapeDtypeStruct((G, 4), jnp.float32),
    )(pooled, graph_attr, d1_W, d1_b, d2_W, d2_b, o_W, o_b)
    return out


# speedup vs baseline: 2.3544x; 1.2166x over previous
import functools

import jax
import jax.numpy as jnp
from jax import lax
from jax.experimental import pallas as pl
from jax.experimental.pallas import tpu as pltpu
from jax.experimental.pallas import tpu_sc as plsc

N = 10000
E = 320000
D = 128
DE = 16
G = 16
NGF = 8
DN = 256
OUT = 4
EPS = 1e-7

NSC = 2
NT = 16
CH = 128
TILE_EDGES = 10240
EP = NSC * NT * TILE_EDGES
PAD = EP - E
ACC_ROWS = 10240
ROWS_PER_TILE = ACC_ROWS // NT
HALF = 64

NBLK = 10
BN = N // NBLK
EBLK = 160
BE = EP // EBLK


def _conv_sc_body(hlo, hhi, ealo, eahi, srcp, dstp, out,
                  acc, src_v, dst_v, g_v, ea_v, upd_v, sem):
    cid = lax.axis_index("c")
    sid = lax.axis_index("s")
    wid = cid * NT + sid
    tile_base = wid * TILE_EDGES

    z16 = jnp.zeros((16,), jnp.float32)

    for half in range(2):
        h_hbm = hlo if half == 0 else hhi
        ea_hbm = ealo if half == 0 else eahi

        def zb(i, c):
            for j in range(8):
                upd_v[i, pl.ds(j * 16, 16)] = z16
            return c
        lax.fori_loop(0, 128, zb, 0)

        def zc(r, c):
            pltpu.sync_copy(upd_v, acc.at[pl.ds(sid * ROWS_PER_TILE + r * 128, 128)])
            return c
        lax.fori_loop(0, ROWS_PER_TILE // 128, zc, 0)
        plsc.subcore_barrier()

        def ck(k, c):
            base = tile_base + k * CH
            pltpu.sync_copy(srcp.at[pl.ds(base, CH)], src_v)
            pltpu.sync_copy(dstp.at[pl.ds(base, CH)], dst_v)
            pltpu.async_copy(h_hbm.at[src_v], g_v, sem).wait()
            pltpu.sync_copy(ea_hbm.at[pl.ds(base, CH)], ea_v)

            def ce(e, cc):
                for j in range(4):
                    g = g_v[e, pl.ds(j * 16, 16)]
                    a = ea_v[e, pl.ds(j * 16, 16)]
                    t = jnp.maximum(g + a, 0.0) + EPS
                    p = jnp.exp(t)
                    upd_v[e, pl.ds(j * 16, 16)] = p
                    upd_v[e, pl.ds(HALF + j * 16, 16)] = t * p
                return cc
            lax.fori_loop(0, CH, ce, 0)
            pltpu.sync_copy(upd_v, acc.at[dst_v], add=True)
            return c
        lax.fori_loop(0, TILE_EDGES // CH, ck, 0)
        plsc.subcore_barrier()

        lo = sid * ROWS_PER_TILE

        @pl.when(sid < NT - 1)
        def _():
            pltpu.sync_copy(acc.at[pl.ds(lo, ROWS_PER_TILE)],
                            out.at[cid, half, pl.ds(lo, ROWS_PER_TILE)])

        @pl.when(sid == NT - 1)
        def _():
            pltpu.sync_copy(acc.at[pl.ds((NT - 1) * ROWS_PER_TILE, N - (NT - 1) * ROWS_PER_TILE)],
                            out.at[cid, half, pl.ds((NT - 1) * ROWS_PER_TILE, N - (NT - 1) * ROWS_PER_TILE)])


_conv_sc = functools.partial(
    pl.kernel,
    out_type=jax.ShapeDtypeStruct((NSC, 2, N, D), jnp.float32),
    mesh=plsc.VectorSubcoreMesh(core_axis_name="c", subcore_axis_name="s"),
    scratch_types=[
        pltpu.VMEM_SHARED((ACC_ROWS, D), jnp.float32),
        pltpu.VMEM((CH,), jnp.int32),
        pltpu.VMEM((CH,), jnp.int32),
        pltpu.VMEM((CH, HALF), jnp.float32),
        pltpu.VMEM((CH, HALF), jnp.float32),
        pltpu.VMEM((CH, D), jnp.float32),
        pltpu.SemaphoreType.DMA,
    ],
    compiler_params=pltpu.CompilerParams(use_tc_tiling_on_sc=False),
)(_conv_sc_body)


def _ea_body(ea_ref, W_ref, b_ref, olo_ref, ohi_ref):
    r = jnp.dot(ea_ref[...], W_ref[...].T, preferred_element_type=jnp.float32) + b_ref[...]
    olo_ref[...] = r[:, :HALF]
    ohi_ref[...] = r[:, HALF:]


_ea_tc = pl.pallas_call(
    _ea_body,
    grid=(EBLK,),
    in_specs=[
        pl.BlockSpec((BE, DE), lambda i: (i, 0)),
        pl.BlockSpec((D, DE), lambda i: (0, 0)),
        pl.BlockSpec((D,), lambda i: (0,)),
    ],
    out_specs=[
        pl.BlockSpec((BE, HALF), lambda i: (i, 0)),
        pl.BlockSpec((BE, HALF), lambda i: (i, 0)),
    ],
    out_shape=[
        jax.ShapeDtypeStruct((EP, HALF), jnp.float32),
        jax.ShapeDtypeStruct((EP, HALF), jnp.float32),
    ],
)


def _h_body(x_ref, W_ref, b_ref, olo_ref, ohi_ref):
    r = jnp.dot(x_ref[...], W_ref[...].T, preferred_element_type=jnp.float32) + b_ref[...]
    olo_ref[...] = r[:, :HALF]
    ohi_ref[...] = r[:, HALF:]


_h_tc = pl.pallas_call(
    _h_body,
    grid=(NBLK,),
    in_specs=[
        pl.BlockSpec((BN, D), lambda i: (i, 0)),
        pl.BlockSpec((D, D), lambda i: (0, 0)),
        pl.BlockSpec((D,), lambda i: (0,)),
    ],
    out_specs=[
        pl.BlockSpec((BN, HALF), lambda i: (i, 0)),
        pl.BlockSpec((BN, HALF), lambda i: (i, 0)),
    ],
    out_shape=[
        jax.ShapeDtypeStruct((N, HALF), jnp.float32),
        jax.ShapeDtypeStruct((N, HALF), jnp.float32),
    ],
)


def _combine_body(p_ref, hlo_ref, hhi_ref, W1_ref, b1_ref, W2_ref, b2_ref,
                  olo_ref, ohi_ref):
    p = p_ref[...]
    s_lo = p[0, 0, :, :HALF] + p[1, 0, :, :HALF]
    w_lo = p[0, 0, :, HALF:] + p[1, 0, :, HALF:]
    s_hi = p[0, 1, :, :HALF] + p[1, 1, :, :HALF]
    w_hi = p[0, 1, :, HALF:] + p[1, 1, :, HALF:]
    out_lo = hlo_ref[...] + w_lo / jnp.maximum(s_lo, 1e-30)
    out_hi = hhi_ref[...] + w_hi / jnp.maximum(s_hi, 1e-30)
    o = jnp.concatenate([out_lo, out_hi], axis=1)
    h1 = jax.nn.relu(jnp.dot(o, W1_ref[...].T, preferred_element_type=jnp.float32) + b1_ref[...])
    h2 = jax.nn.relu(jnp.dot(h1, W2_ref[...].T, preferred_element_type=jnp.float32) + b2_ref[...])
    olo_ref[...] = h2[:, :HALF]
    ohi_ref[...] = h2[:, HALF:]


_combine_tc = pl.pallas_call(
    _combine_body,
    grid=(NBLK,),
    in_specs=[
        pl.BlockSpec((NSC, 2, BN, D), lambda i: (0, 0, i, 0)),
        pl.BlockSpec((BN, HALF), lambda i: (i, 0)),
        pl.BlockSpec((BN, HALF), lambda i: (i, 0)),
        pl.BlockSpec((2 * D, D), lambda i: (0, 0)),
        pl.BlockSpec((2 * D,), lambda i: (0,)),
        pl.BlockSpec((D, 2 * D), lambda i: (0, 0)),
        pl.BlockSpec((D,), lambda i: (0,)),
    ],
    out_specs=[
        pl.BlockSpec((BN, HALF), lambda i: (i, 0)),
        pl.BlockSpec((BN, HALF), lambda i: (i, 0)),
    ],
    out_shape=[
        jax.ShapeDtypeStruct((N, HALF), jnp.float32),
        jax.ShapeDtypeStruct((N, HALF), jnp.float32),
    ],
)


def _head_body(hlo_ref, hhi_ref, b_ref, ga_ref, d1W_ref, d1b_ref, d2W_ref,
               d2b_ref, oW_ref, ob_ref, out_ref, pooled, cnt):
    i = pl.program_id(0)

    @pl.when(i == 0)
    def _():
        pooled[...] = jnp.zeros_like(pooled)
        cnt[...] = jnp.zeros_like(cnt)

    b = b_ref[0, 0, :]
    oh = (b[None, :] == lax.broadcasted_iota(jnp.int32, (G, BN), 0).astype(jnp.float32)).astype(jnp.float32)
    hblk = jnp.concatenate([hlo_ref[...], hhi_ref[...]], axis=1)
    pooled[...] += jnp.dot(oh, hblk, preferred_element_type=jnp.float32)
    cnt[...] += jnp.dot(oh, jnp.ones_like(hblk), preferred_element_type=jnp.float32)

    @pl.when(i == pl.num_programs(0) - 1)
    def _():
        pm = pooled[...] / jnp.maximum(cnt[...], 1.0)
        g = jnp.concatenate([pm, ga_ref[...]], axis=1)
        g = jax.nn.relu(jnp.dot(g, d1W_ref[...].T, preferred_element_type=jnp.float32) + d1b_ref[...])
        g = jax.nn.relu(jnp.dot(g, d2W_ref[...].T, preferred_element_type=jnp.float32) + d2b_ref[...])
        out_ref[...] = jax.nn.sigmoid(jnp.dot(g, oW_ref[...].T, preferred_element_type=jnp.float32) + ob_ref[...])


_head_tc = pl.pallas_call(
    _head_body,
    grid=(NBLK,),
    in_specs=[
        pl.BlockSpec((BN, HALF), lambda i: (i, 0)),
        pl.BlockSpec((BN, HALF), lambda i: (i, 0)),
        pl.BlockSpec((1, 1, BN), lambda i: (i, 0, 0)),
        pl.BlockSpec((G, NGF), lambda i: (0, 0)),
        pl.BlockSpec((DN, D + NGF), lambda i: (0, 0)),
        pl.BlockSpec((DN,), lambda i: (0,)),
        pl.BlockSpec((DN, DN), lambda i: (0, 0)),
        pl.BlockSpec((DN,), lambda i: (0,)),
        pl.BlockSpec((OUT, DN), lambda i: (0, 0)),
        pl.BlockSpec((OUT,), lambda i: (0,)),
    ],
    out_specs=pl.BlockSpec((G, OUT), lambda i: (0, 0)),
    out_shape=jax.ShapeDtypeStruct((G, OUT), jnp.float32),
    scratch_shapes=[
        pltpu.VMEM((G, D), jnp.float32),
        pltpu.VMEM((G, D), jnp.float32),
    ],
)


def kernel(x, edge_index, edge_attr, graph_attr, batch, node_W, node_b,
           edge_W, edge_b, c1_W1, c1_b1, c1_W2, c1_b2, c2_W1, c2_b1, c2_W2,
           c2_b2, c3_W1, c3_b1, c3_W2, c3_b2, d1_W, d1_b, d2_W, d2_b, o_W, o_b):
    src = edge_index[0]
    dst = edge_index[1]
    ar = jnp.arange(PAD, dtype=jnp.int32)
    srcp = jnp.concatenate([src, (ar * 37) % N])
    dstp = jnp.concatenate([dst, N + (ar % (ACC_ROWS - N))])
    eap = jnp.concatenate([edge_attr, jnp.zeros((PAD, DE), jnp.float32)])
    batch_r = batch.astype(jnp.float32).reshape(NBLK, 1, BN)

    ealo, eahi = _ea_tc(eap, edge_W, edge_b)
    hlo, hhi = _h_tc(x, node_W, node_b)
    for W1, b1, W2, b2 in ((c1_W1, c1_b1, c1_W2, c1_b2),
                           (c2_W1, c2_b1, c2_W2, c2_b2),
                           (c3_W1, c3_b1, c3_W2, c3_b2)):
        part = _conv_sc(hlo, hhi, ealo, eahi, srcp, dstp)
        hlo, hhi = _combine_tc(part, hlo, hhi, W1, b1, W2, b2)
    return _head_tc(hlo, hhi, batch_r, graph_attr, d1_W, d1_b, d2_W, d2_b,
                    o_W, o_b)


# speedup vs baseline: 9.5474x; 4.0552x over previous
import functools

import jax
import jax.numpy as jnp
from jax import lax
from jax.experimental import pallas as pl
from jax.experimental.pallas import tpu as pltpu
from jax.experimental.pallas import tpu_sc as plsc

N = 10000
E = 320000
D = 128
DE = 16
G = 16
NGF = 8
DN = 256
OUT = 4
EPS = 1e-7

NSC = 2
NT = 16
CH = 64
TILE_EDGES = 10240
EP = NSC * NT * TILE_EDGES
PAD = EP - E
ACC_ROWS = 10240
ROWS_PER_TILE = ACC_ROWS // NT
HALF = 64
CHUNKS = TILE_EDGES // CH
BULK = CHUNKS // 2
NPAIR = BULK // 2

NBLK = 10
BN = N // NBLK
EBLK = 160
BE = EP // EBLK


def _conv_sc_body(hlo, hhi, ealo, eahi, srcp, dstp, out,
                  acc, src_all, dst_all, g_v, ea_v, upd_v,
                  sg0, sg1, se0, se1, ss0, ss1):
    cid = lax.axis_index("c")
    sid = lax.axis_index("s")
    wid = cid * NT + sid

    sgs = (sg0, sg1)
    ses = (se0, se1)
    sss = (ss0, ss1)
    z16 = jnp.zeros((16,), jnp.float32)

    for half in range(2):
        h_hbm = hlo if half == 0 else hhi
        ea_hbm = ealo if half == 0 else eahi

        def zb(i, c):
            for j in range(8):
                upd_v[0, i, pl.ds(j * 16, 16)] = z16
            return c
        lax.fori_loop(0, CH, zb, 0)

        def zc(r, c):
            pltpu.sync_copy(upd_v.at[0], acc.at[pl.ds(sid * ROWS_PER_TILE + r * CH, CH)])
            return c
        lax.fori_loop(0, ROWS_PER_TILE // CH, zc, 0)
        plsc.subcore_barrier()

        for bulk in range(2):
            row0 = wid * CHUNKS + bulk * BULK
            pltpu.sync_copy(srcp.at[pl.ds(row0, BULK)], src_all)
            pltpu.sync_copy(dstp.at[pl.ds(row0, BULK)], dst_all)

            def fetch(c, slot):
                pltpu.async_copy(h_hbm.at[src_all.at[c]], g_v.at[slot], sgs[slot])
                pltpu.async_copy(ea_hbm.at[pl.ds((row0 + c) * CH, CH)], ea_v.at[slot], ses[slot])

            def wait_fetch(slot):
                pltpu.make_async_copy(h_hbm.at[src_all.at[0]], g_v.at[slot], sgs[slot]).wait()
                pltpu.make_async_copy(ea_hbm.at[pl.ds(0, CH)], ea_v.at[slot], ses[slot]).wait()

            def compute(slot):
                @plsc.parallel_loop(0, CH, step=1, unroll=4)
                def _(e):
                    for j in range(4):
                        gv = g_v[slot, e, pl.ds(j * 16, 16)]
                        av = ea_v[slot, e, pl.ds(j * 16, 16)]
                        t = jnp.maximum(gv + av, 0.0) + EPS
                        p = jnp.exp(t)
                        upd_v[slot, e, pl.ds(j * 16, 16)] = p
                        upd_v[slot, e, pl.ds(HALF + j * 16, 16)] = t * p

            def scatter(c, slot):
                pltpu.async_copy(upd_v.at[slot], acc.at[dst_all.at[c]], sss[slot], add=True)

            def wait_scatter(slot):
                pltpu.make_async_copy(upd_v.at[slot], acc.at[dst_all.at[0]], sss[slot]).wait()

            fetch(0, 0)

            def pair(it, c):
                c0 = it * 2
                fetch(c0 + 1, 1)
                wait_fetch(0)

                @pl.when(it > 0)
                def _():
                    wait_scatter(0)
                compute(0)
                scatter(c0, 0)

                @pl.when(it < NPAIR - 1)
                def _():
                    fetch(c0 + 2, 0)
                wait_fetch(1)

                @pl.when(it > 0)
                def _():
                    wait_scatter(1)
                compute(1)
                scatter(c0 + 1, 1)
                return c
            lax.fori_loop(0, NPAIR, pair, 0)
            wait_scatter(0)
            wait_scatter(1)
        plsc.subcore_barrier()

        lo = sid * ROWS_PER_TILE

        @pl.when(sid < NT - 1)
        def _():
            pltpu.sync_copy(acc.at[pl.ds(lo, ROWS_PER_TILE)],
                            out.at[cid, half, pl.ds(lo, ROWS_PER_TILE)])

        @pl.when(sid == NT - 1)
        def _():
            pltpu.sync_copy(acc.at[pl.ds((NT - 1) * ROWS_PER_TILE, N - (NT - 1) * ROWS_PER_TILE)],
                            out.at[cid, half, pl.ds((NT - 1) * ROWS_PER_TILE, N - (NT - 1) * ROWS_PER_TILE)])


_conv_sc = functools.partial(
    pl.kernel,
    out_type=jax.ShapeDtypeStruct((NSC, 2, N, D), jnp.float32),
    mesh=plsc.VectorSubcoreMesh(core_axis_name="c", subcore_axis_name="s"),
    scratch_types=[
        pltpu.VMEM_SHARED((ACC_ROWS, D), jnp.float32),
        pltpu.VMEM((BULK, CH), jnp.int32),
        pltpu.VMEM((BULK, CH), jnp.int32),
        pltpu.VMEM((2, CH, HALF), jnp.float32),
        pltpu.VMEM((2, CH, HALF), jnp.float32),
        pltpu.VMEM((2, CH, D), jnp.float32),
        pltpu.SemaphoreType.DMA,
        pltpu.SemaphoreType.DMA,
        pltpu.SemaphoreType.DMA,
        pltpu.SemaphoreType.DMA,
        pltpu.SemaphoreType.DMA,
        pltpu.SemaphoreType.DMA,
    ],
    compiler_params=pltpu.CompilerParams(use_tc_tiling_on_sc=False),
)(_conv_sc_body)


def _ea_body(ea_ref, W_ref, b_ref, olo_ref, ohi_ref):
    r = jnp.dot(ea_ref[...], W_ref[...].T, preferred_element_type=jnp.float32) + b_ref[...]
    olo_ref[...] = r[:, :HALF]
    ohi_ref[...] = r[:, HALF:]


_ea_tc = pl.pallas_call(
    _ea_body,
    grid=(EBLK,),
    in_specs=[
        pl.BlockSpec((BE, DE), lambda i: (i, 0)),
        pl.BlockSpec((D, DE), lambda i: (0, 0)),
        pl.BlockSpec((D,), lambda i: (0,)),
    ],
    out_specs=[
        pl.BlockSpec((BE, HALF), lambda i: (i, 0)),
        pl.BlockSpec((BE, HALF), lambda i: (i, 0)),
    ],
    out_shape=[
        jax.ShapeDtypeStruct((EP, HALF), jnp.float32),
        jax.ShapeDtypeStruct((EP, HALF), jnp.float32),
    ],
)


def _h_body(x_ref, W_ref, b_ref, olo_ref, ohi_ref):
    r = jnp.dot(x_ref[...], W_ref[...].T, preferred_element_type=jnp.float32) + b_ref[...]
    olo_ref[...] = r[:, :HALF]
    ohi_ref[...] = r[:, HALF:]


_h_tc = pl.pallas_call(
    _h_body,
    grid=(NBLK,),
    in_specs=[
        pl.BlockSpec((BN, D), lambda i: (i, 0)),
        pl.BlockSpec((D, D), lambda i: (0, 0)),
        pl.BlockSpec((D,), lambda i: (0,)),
    ],
    out_specs=[
        pl.BlockSpec((BN, HALF), lambda i: (i, 0)),
        pl.BlockSpec((BN, HALF), lambda i: (i, 0)),
    ],
    out_shape=[
        jax.ShapeDtypeStruct((N, HALF), jnp.float32),
        jax.ShapeDtypeStruct((N, HALF), jnp.float32),
    ],
)


def _combine_body(p_ref, hlo_ref, hhi_ref, W1_ref, b1_ref, W2_ref, b2_ref,
                  olo_ref, ohi_ref):
    p = p_ref[...]
    s_lo = p[0, 0, :, :HALF] + p[1, 0, :, :HALF]
    w_lo = p[0, 0, :, HALF:] + p[1, 0, :, HALF:]
    s_hi = p[0, 1, :, :HALF] + p[1, 1, :, :HALF]
    w_hi = p[0, 1, :, HALF:] + p[1, 1, :, HALF:]
    out_lo = hlo_ref[...] + w_lo / jnp.maximum(s_lo, 1e-30)
    out_hi = hhi_ref[...] + w_hi / jnp.maximum(s_hi, 1e-30)
    o = jnp.concatenate([out_lo, out_hi], axis=1)
    h1 = jax.nn.relu(jnp.dot(o, W1_ref[...].T, preferred_element_type=jnp.float32) + b1_ref[...])
    h2 = jax.nn.relu(jnp.dot(h1, W2_ref[...].T, preferred_element_type=jnp.float32) + b2_ref[...])
    olo_ref[...] = h2[:, :HALF]
    ohi_ref[...] = h2[:, HALF:]


_combine_tc = pl.pallas_call(
    _combine_body,
    grid=(NBLK,),
    in_specs=[
        pl.BlockSpec((NSC, 2, BN, D), lambda i: (0, 0, i, 0)),
        pl.BlockSpec((BN, HALF), lambda i: (i, 0)),
        pl.BlockSpec((BN, HALF), lambda i: (i, 0)),
        pl.BlockSpec((2 * D, D), lambda i: (0, 0)),
        pl.BlockSpec((2 * D,), lambda i: (0,)),
        pl.BlockSpec((D, 2 * D), lambda i: (0, 0)),
        pl.BlockSpec((D,), lambda i: (0,)),
    ],
    out_specs=[
        pl.BlockSpec((BN, HALF), lambda i: (i, 0)),
        pl.BlockSpec((BN, HALF), lambda i: (i, 0)),
    ],
    out_shape=[
        jax.ShapeDtypeStruct((N, HALF), jnp.float32),
        jax.ShapeDtypeStruct((N, HALF), jnp.float32),
    ],
)


def _head_body(hlo_ref, hhi_ref, b_ref, ga_ref, d1W_ref, d1b_ref, d2W_ref,
               d2b_ref, oW_ref, ob_ref, out_ref, pooled, cnt):
    i = pl.program_id(0)

    @pl.when(i == 0)
    def _():
        pooled[...] = jnp.zeros_like(pooled)
        cnt[...] = jnp.zeros_like(cnt)

    b = b_ref[0, 0, :]
    oh = (b[None, :] == lax.broadcasted_iota(jnp.int32, (G, BN), 0).astype(jnp.float32)).astype(jnp.float32)
    hblk = jnp.concatenate([hlo_ref[...], hhi_ref[...]], axis=1)
    pooled[...] += jnp.dot(oh, hblk, preferred_element_type=jnp.float32)
    cnt[...] += jnp.dot(oh, jnp.ones_like(hblk), preferred_element_type=jnp.float32)

    @pl.when(i == pl.num_programs(0) - 1)
    def _():
        pm = pooled[...] / jnp.maximum(cnt[...], 1.0)
        g = jnp.concatenate([pm, ga_ref[...]], axis=1)
        g = jax.nn.relu(jnp.dot(g, d1W_ref[...].T, preferred_element_type=jnp.float32) + d1b_ref[...])
        g = jax.nn.relu(jnp.dot(g, d2W_ref[...].T, preferred_element_type=jnp.float32) + d2b_ref[...])
        out_ref[...] = jax.nn.sigmoid(jnp.dot(g, oW_ref[...].T, preferred_element_type=jnp.float32) + ob_ref[...])


_head_tc = pl.pallas_call(
    _head_body,
    grid=(NBLK,),
    in_specs=[
        pl.BlockSpec((BN, HALF), lambda i: (i, 0)),
        pl.BlockSpec((BN, HALF), lambda i: (i, 0)),
        pl.BlockSpec((1, 1, BN), lambda i: (i, 0, 0)),
        pl.BlockSpec((G, NGF), lambda i: (0, 0)),
        pl.BlockSpec((DN, D + NGF), lambda i: (0, 0)),
        pl.BlockSpec((DN,), lambda i: (0,)),
        pl.BlockSpec((DN, DN), lambda i: (0, 0)),
        pl.BlockSpec((DN,), lambda i: (0,)),
        pl.BlockSpec((OUT, DN), lambda i: (0, 0)),
        pl.BlockSpec((OUT,), lambda i: (0,)),
    ],
    out_specs=pl.BlockSpec((G, OUT), lambda i: (0, 0)),
    out_shape=jax.ShapeDtypeStruct((G, OUT), jnp.float32),
    scratch_shapes=[
        pltpu.VMEM((G, D), jnp.float32),
        pltpu.VMEM((G, D), jnp.float32),
    ],
)


def kernel(x, edge_index, edge_attr, graph_attr, batch, node_W, node_b,
           edge_W, edge_b, c1_W1, c1_b1, c1_W2, c1_b2, c2_W1, c2_b1, c2_W2,
           c2_b2, c3_W1, c3_b1, c3_W2, c3_b2, d1_W, d1_b, d2_W, d2_b, o_W, o_b):
    src = edge_index[0]
    dst = edge_index[1]
    ar = jnp.arange(PAD, dtype=jnp.int32)
    srcp = jnp.concatenate([src, (ar * 37) % N]).reshape(EP // CH, CH)
    dstp = jnp.concatenate([dst, N + (ar % (ACC_ROWS - N))]).reshape(EP // CH, CH)
    eap = jnp.concatenate([edge_attr, jnp.zeros((PAD, DE), jnp.float32)])
    batch_r = batch.astype(jnp.float32).reshape(NBLK, 1, BN)

    ealo, eahi = _ea_tc(eap, edge_W, edge_b)
    hlo, hhi = _h_tc(x, node_W, node_b)
    for W1, b1, W2, b2 in ((c1_W1, c1_b1, c1_W2, c1_b2),
                           (c2_W1, c2_b1, c2_W2, c2_b2),
                           (c3_W1, c3_b1, c3_W2, c3_b2)):
        part = _conv_sc(hlo, hhi, ealo, eahi, srcp, dstp)
        hlo, hhi = _combine_tc(part, hlo, hhi, W1, b1, W2, b2)
    return _head_tc(hlo, hhi, batch_r, graph_attr, d1_W, d1_b, d2_W, d2_b,
                    o_W, o_b)


# speedup vs baseline: 12.4447x; 1.3035x over previous
import functools

import jax
import jax.numpy as jnp
from jax import lax
from jax.experimental import pallas as pl
from jax.experimental.pallas import tpu as pltpu
from jax.experimental.pallas import tpu_sc as plsc

N = 10000
E = 320000
D = 128
DE = 16
G = 16
NGF = 8
DN = 256
OUT = 4
EPS = 1e-7

NSC = 2
NT = 16
CH = 64
TILE_EDGES = 10240
EP = NSC * NT * TILE_EDGES
PAD = EP - E
ACC_ROWS = 10240
ROWS_PER_TILE = ACC_ROWS // NT
HALF = 64
CHUNKS = TILE_EDGES // CH
BULK = CHUNKS // 2
NPAIR = BULK // 2

NBLK = 10
BN = N // NBLK
BE = 2560
EBLK = EP // BE


def _conv_sc_body(hlo, hhi, ea_hbm, srcp, dstp, out,
                  acc, src_all, dst_all, g_v, ea_v, upd_v,
                  sg0, sg1, se0, se1, ss0, ss1):
    cid = lax.axis_index("c")
    sid = lax.axis_index("s")
    wid = cid * NT + sid

    sgs = (sg0, sg1)
    ses = (se0, se1)
    sss = (ss0, ss1)
    z16 = jnp.zeros((16,), jnp.float32)

    for half in range(2):
        h_hbm = hlo if half == 0 else hhi

        def zb(i, c):
            for j in range(8):
                upd_v[0, i, pl.ds(j * 16, 16)] = z16
            return c
        lax.fori_loop(0, CH, zb, 0)

        def zc(r, c):
            pltpu.sync_copy(upd_v.at[0], acc.at[pl.ds(sid * ROWS_PER_TILE + r * CH, CH)])
            return c
        lax.fori_loop(0, ROWS_PER_TILE // CH, zc, 0)
        plsc.subcore_barrier()

        for bulk in range(2):
            row0 = wid * CHUNKS + bulk * BULK
            pltpu.sync_copy(srcp.at[pl.ds(row0, BULK)], src_all)
            pltpu.sync_copy(dstp.at[pl.ds(row0, BULK)], dst_all)

            def fetch(c, slot):
                pltpu.async_copy(h_hbm.at[src_all.at[c]], g_v.at[slot], sgs[slot])
                pltpu.async_copy(
                    ea_hbm.at[pl.ds((row0 + c) * CH, CH), pl.ds(half * HALF, HALF)],
                    ea_v.at[slot], ses[slot])

            def wait_fetch(slot):
                pltpu.make_async_copy(h_hbm.at[src_all.at[0]], g_v.at[slot], sgs[slot]).wait()
                pltpu.make_async_copy(
                    ea_hbm.at[pl.ds(0, CH), pl.ds(half * HALF, HALF)],
                    ea_v.at[slot], ses[slot]).wait()

            def compute(slot):
                @plsc.parallel_loop(0, CH, step=1, unroll=4)
                def _(e):
                    for j in range(4):
                        gv = g_v[slot, e, pl.ds(j * 16, 16)]
                        av = ea_v[slot, e, pl.ds(j * 16, 16)]
                        t = jnp.maximum(gv + av, 0.0) + EPS
                        p = jnp.exp(t)
                        upd_v[slot, e, pl.ds(j * 16, 16)] = p
                        upd_v[slot, e, pl.ds(HALF + j * 16, 16)] = t * p

            def scatter(c, slot):
                pltpu.async_copy(upd_v.at[slot], acc.at[dst_all.at[c]], sss[slot], add=True)

            def wait_scatter(slot):
                pltpu.make_async_copy(upd_v.at[slot], acc.at[dst_all.at[0]], sss[slot]).wait()

            fetch(0, 0)

            def pair(it, c):
                c0 = it * 2
                fetch(c0 + 1, 1)
                wait_fetch(0)

                @pl.when(it > 0)
                def _():
                    wait_scatter(0)
                compute(0)
                scatter(c0, 0)

                @pl.when(it < NPAIR - 1)
                def _():
                    fetch(c0 + 2, 0)
                wait_fetch(1)

                @pl.when(it > 0)
                def _():
                    wait_scatter(1)
                compute(1)
                scatter(c0 + 1, 1)
                return c
            lax.fori_loop(0, NPAIR, pair, 0)
            wait_scatter(0)
            wait_scatter(1)
        plsc.subcore_barrier()

        lo = sid * ROWS_PER_TILE

        @pl.when(sid < NT - 1)
        def _():
            pltpu.sync_copy(acc.at[pl.ds(lo, ROWS_PER_TILE)],
                            out.at[cid, half, pl.ds(lo, ROWS_PER_TILE)])

        @pl.when(sid == NT - 1)
        def _():
            pltpu.sync_copy(acc.at[pl.ds((NT - 1) * ROWS_PER_TILE, N - (NT - 1) * ROWS_PER_TILE)],
                            out.at[cid, half, pl.ds((NT - 1) * ROWS_PER_TILE, N - (NT - 1) * ROWS_PER_TILE)])


_conv_sc = functools.partial(
    pl.kernel,
    out_type=jax.ShapeDtypeStruct((NSC, 2, N, D), jnp.float32),
    mesh=plsc.VectorSubcoreMesh(core_axis_name="c", subcore_axis_name="s"),
    scratch_types=[
        pltpu.VMEM_SHARED((ACC_ROWS, D), jnp.float32),
        pltpu.VMEM((BULK, CH), jnp.int32),
        pltpu.VMEM((BULK, CH), jnp.int32),
        pltpu.VMEM((2, CH, HALF), jnp.float32),
        pltpu.VMEM((2, CH, HALF), jnp.float32),
        pltpu.VMEM((2, CH, D), jnp.float32),
        pltpu.SemaphoreType.DMA,
        pltpu.SemaphoreType.DMA,
        pltpu.SemaphoreType.DMA,
        pltpu.SemaphoreType.DMA,
        pltpu.SemaphoreType.DMA,
        pltpu.SemaphoreType.DMA,
    ],
    compiler_params=pltpu.CompilerParams(use_tc_tiling_on_sc=False),
)(_conv_sc_body)


def _ea_body(ea_ref, W_ref, b_ref, o_ref):
    o_ref[...] = jnp.dot(ea_ref[...], W_ref[...].T, preferred_element_type=jnp.float32) + b_ref[...]


_ea_tc = pl.pallas_call(
    _ea_body,
    grid=(EBLK,),
    in_specs=[
        pl.BlockSpec((BE, DE), lambda i: (jnp.minimum(i, E // BE - 1), 0)),
        pl.BlockSpec((D, DE), lambda i: (0, 0)),
        pl.BlockSpec((D,), lambda i: (0,)),
    ],
    out_specs=pl.BlockSpec((BE, D), lambda i: (i, 0)),
    out_shape=jax.ShapeDtypeStruct((EP, D), jnp.float32),
)


def _h_body(x_ref, W_ref, b_ref, olo_ref, ohi_ref):
    r = jnp.dot(x_ref[...], W_ref[...].T, preferred_element_type=jnp.float32) + b_ref[...]
    olo_ref[...] = r[:, :HALF]
    ohi_ref[...] = r[:, HALF:]


_h_tc = pl.pallas_call(
    _h_body,
    grid=(NBLK,),
    in_specs=[
        pl.BlockSpec((BN, D), lambda i: (i, 0)),
        pl.BlockSpec((D, D), lambda i: (0, 0)),
        pl.BlockSpec((D,), lambda i: (0,)),
    ],
    out_specs=[
        pl.BlockSpec((BN, HALF), lambda i: (i, 0)),
        pl.BlockSpec((BN, HALF), lambda i: (i, 0)),
    ],
    out_shape=[
        jax.ShapeDtypeStruct((N, HALF), jnp.float32),
        jax.ShapeDtypeStruct((N, HALF), jnp.float32),
    ],
)


def _combine_body(p_ref, hlo_ref, hhi_ref, W1_ref, b1_ref, W2_ref, b2_ref,
                  olo_ref, ohi_ref):
    p = p_ref[...]
    s_lo = p[0, 0, :, :HALF] + p[1, 0, :, :HALF]
    w_lo = p[0, 0, :, HALF:] + p[1, 0, :, HALF:]
    s_hi = p[0, 1, :, :HALF] + p[1, 1, :, :HALF]
    w_hi = p[0, 1, :, HALF:] + p[1, 1, :, HALF:]
    out_lo = hlo_ref[...] + w_lo / jnp.maximum(s_lo, 1e-30)
    out_hi = hhi_ref[...] + w_hi / jnp.maximum(s_hi, 1e-30)
    o = jnp.concatenate([out_lo, out_hi], axis=1)
    h1 = jax.nn.relu(jnp.dot(o, W1_ref[...].T, preferred_element_type=jnp.float32) + b1_ref[...])
    h2 = jax.nn.relu(jnp.dot(h1, W2_ref[...].T, preferred_element_type=jnp.float32) + b2_ref[...])
    olo_ref[...] = h2[:, :HALF]
    ohi_ref[...] = h2[:, HALF:]


_combine_tc = pl.pallas_call(
    _combine_body,
    grid=(NBLK,),
    in_specs=[
        pl.BlockSpec((NSC, 2, BN, D), lambda i: (0, 0, i, 0)),
        pl.BlockSpec((BN, HALF), lambda i: (i, 0)),
        pl.BlockSpec((BN, HALF), lambda i: (i, 0)),
        pl.BlockSpec((2 * D, D), lambda i: (0, 0)),
        pl.BlockSpec((2 * D,), lambda i: (0,)),
        pl.BlockSpec((D, 2 * D), lambda i: (0, 0)),
        pl.BlockSpec((D,), lambda i: (0,)),
    ],
    out_specs=[
        pl.BlockSpec((BN, HALF), lambda i: (i, 0)),
        pl.BlockSpec((BN, HALF), lambda i: (i, 0)),
    ],
    out_shape=[
        jax.ShapeDtypeStruct((N, HALF), jnp.float32),
        jax.ShapeDtypeStruct((N, HALF), jnp.float32),
    ],
)


def _head_body(hlo_ref, hhi_ref, b_ref, ga_ref, d1W_ref, d1b_ref, d2W_ref,
               d2b_ref, oW_ref, ob_ref, out_ref, pooled, cnt):
    i = pl.program_id(0)

    @pl.when(i == 0)
    def _():
        pooled[...] = jnp.zeros_like(pooled)
        cnt[...] = jnp.zeros_like(cnt)

    b = b_ref[0, 0, :]
    oh = (b[None, :] == lax.broadcasted_iota(jnp.int32, (G, BN), 0).astype(jnp.float32)).astype(jnp.float32)
    hblk = jnp.concatenate([hlo_ref[...], hhi_ref[...]], axis=1)
    pooled[...] += jnp.dot(oh, hblk, preferred_element_type=jnp.float32)
    cnt[...] += jnp.dot(oh, jnp.ones_like(hblk), preferred_element_type=jnp.float32)

    @pl.when(i == pl.num_programs(0) - 1)
    def _():
        pm = pooled[...] / jnp.maximum(cnt[...], 1.0)
        g = jnp.concatenate([pm, ga_ref[...]], axis=1)
        g = jax.nn.relu(jnp.dot(g, d1W_ref[...].T, preferred_element_type=jnp.float32) + d1b_ref[...])
        g = jax.nn.relu(jnp.dot(g, d2W_ref[...].T, preferred_element_type=jnp.float32) + d2b_ref[...])
        out_ref[...] = jax.nn.sigmoid(jnp.dot(g, oW_ref[...].T, preferred_element_type=jnp.float32) + ob_ref[...])


_head_tc = pl.pallas_call(
    _head_body,
    grid=(NBLK,),
    in_specs=[
        pl.BlockSpec((BN, HALF), lambda i: (i, 0)),
        pl.BlockSpec((BN, HALF), lambda i: (i, 0)),
        pl.BlockSpec((1, 1, BN), lambda i: (i, 0, 0)),
        pl.BlockSpec((G, NGF), lambda i: (0, 0)),
        pl.BlockSpec((DN, D + NGF), lambda i: (0, 0)),
        pl.BlockSpec((DN,), lambda i: (0,)),
        pl.BlockSpec((DN, DN), lambda i: (0, 0)),
        pl.BlockSpec((DN,), lambda i: (0,)),
        pl.BlockSpec((OUT, DN), lambda i: (0, 0)),
        pl.BlockSpec((OUT,), lambda i: (0,)),
    ],
    out_specs=pl.BlockSpec((G, OUT), lambda i: (0, 0)),
    out_shape=jax.ShapeDtypeStruct((G, OUT), jnp.float32),
    scratch_shapes=[
        pltpu.VMEM((G, D), jnp.float32),
        pltpu.VMEM((G, D), jnp.float32),
    ],
)


def kernel(x, edge_index, edge_attr, graph_attr, batch, node_W, node_b,
           edge_W, edge_b, c1_W1, c1_b1, c1_W2, c1_b2, c2_W1, c2_b1, c2_W2,
           c2_b2, c3_W1, c3_b1, c3_W2, c3_b2, d1_W, d1_b, d2_W, d2_b, o_W, o_b):
    src = edge_index[0]
    dst = edge_index[1]
    ar = jnp.arange(PAD, dtype=jnp.int32)
    srcp = jnp.concatenate([src, (ar * 37) % N]).reshape(EP // CH, CH)
    dstp = jnp.concatenate([dst, N + (ar % (ACC_ROWS - N))]).reshape(EP // CH, CH)
    batch_r = batch.astype(jnp.float32).reshape(NBLK, 1, BN)

    ea = _ea_tc(edge_attr, edge_W, edge_b)
    hlo, hhi = _h_tc(x, node_W, node_b)
    for W1, b1, W2, b2 in ((c1_W1, c1_b1, c1_W2, c1_b2),
                           (c2_W1, c2_b1, c2_W2, c2_b2),
                           (c3_W1, c3_b1, c3_W2, c3_b2)):
        part = _conv_sc(hlo, hhi, ea, srcp, dstp)
        hlo, hhi = _combine_tc(part, hlo, hhi, W1, b1, W2, b2)
    return _head_tc(hlo, hhi, batch_r, graph_attr, d1_W, d1_b, d2_W, d2_b,
                    o_W, o_b)
